# Initial kernel scaffold; baseline (speedup 1.0000x reference)
import jax
import jax.numpy as jnp
from jax.experimental import pallas as pl


def kernel(Y, D):
    raise NotImplementedError("write your pallas kernel here")


# speedup vs baseline: 4.0826x; 4.0826x over previous
import functools

import jax
import jax.numpy as jnp
from jax import lax
from jax.experimental import pallas as pl
from jax.experimental.pallas import tpu as pltpu
from jax.experimental.pallas import tpu_sc as plsc

NF = 512
NA = 2048
BATCH_N = 2048
KMAX = 12
DIAG_EPS = 1e-4

NW = 32
PER_W = BATCH_N // NW

BM = 128
BN = 128


def _mm_body(a_ref, dt_ref, o_ref):
    i = pl.program_id(0)
    j = pl.program_id(1)
    acc = jnp.dot(a_ref[...], dt_ref[...], preferred_element_type=jnp.float32)
    r = lax.broadcasted_iota(jnp.int32, (BM, BN), 0)
    c = lax.broadcasted_iota(jnp.int32, (BM, BN), 1)
    acc = acc + jnp.where((i == j) & (r == c), jnp.float32(DIAG_EPS),
                          jnp.float32(0.0))
    o_ref[...] = acc


_mm = pl.pallas_call(
    _mm_body,
    grid=((NA + BATCH_N) // BM, NA // BN),
    in_specs=[
        pl.BlockSpec((BM, NF), lambda i, j: (i, 0)),
        pl.BlockSpec((NF, BN), lambda i, j: (0, j)),
    ],
    out_specs=pl.BlockSpec((BM, BN), lambda i, j: (i, j)),
    out_shape=jax.ShapeDtypeStruct((NA + BATCH_N, NA), jnp.float32),
)


_GDN = lax.GatherDimensionNumbers(
    offset_dims=(), collapsed_slice_dims=(0,), start_index_map=(0,))


def _shuffle(v, perm):
    return lax.gather(v, perm[:, None], _GDN, (1,),
                      mode=lax.GatherScatterMode.PROMISE_IN_BOUNDS)


def _red(v, op):
    lane = lax.iota(jnp.int32, 16)
    for sh in (1, 2, 4, 8):
        v = op(v, _shuffle(v, lane ^ sh))
    return v


def _bf16_round(v):
    b = lax.bitcast_convert_type(v, jnp.int32)
    lsb = lax.shift_right_logical(b, 16) & jnp.int32(1)
    r = (b + jnp.int32(0x7FFF) + lsb) & jnp.int32(-65536)
    return lax.bitcast_convert_type(r, jnp.float32)


def _newton_sqrt(a):
    x = (a + jnp.float32(1.0)) * jnp.float32(0.5)
    for _ in range(12):
        x = jnp.float32(0.5) * (x + a / x)
    return x


@functools.partial(
    pl.kernel,
    out_type=[
        jax.ShapeDtypeStruct((BATCH_N, NA), jnp.float32),
        jax.ShapeDtypeStruct((BATCH_N, NF), jnp.float32),
    ],
    mesh=plsc.VectorSubcoreMesh(core_axis_name="c", subcore_axis_name="s"),
    compiler_params=pltpu.CompilerParams(needs_layout_passes=False),
    scratch_types=[
        pltpu.VMEM((1, NA), jnp.float32),
        [pltpu.VMEM((1, NA), jnp.float32)] * KMAX,
        [pltpu.VMEM((1, NA), jnp.float32)] * KMAX,
        pltpu.VMEM((NA,), jnp.float32),
        pltpu.VMEM((NA,), jnp.float32),
        pltpu.VMEM((1, NA), jnp.float32),
        pltpu.VMEM((16, NF), jnp.float32),
        pltpu.VMEM((1, NF), jnp.float32),
        pltpu.VMEM((1,), jnp.int32),
        pltpu.VMEM((16,), jnp.int32),
        pltpu.SemaphoreType.DMA,
    ],
)
def _omp_sc(hb_hbm, g_hbm, diag_hbm, d_hbm, x_hbm, yp_hbm,
            hb_v, rows_v, rrows_v, mv_v, diag_v, xrow_v, drows_v, yp_v,
            idx1_v, idx16_v, gsem):
    wid = lax.axis_index("s") * 2 + lax.axis_index("c")
    base = wid * PER_W
    lane = lax.iota(jnp.int32, 16)
    pltpu.sync_copy(diag_hbm, diag_v)

    def per_signal(s, carry):
        sig = base + s
        pltpu.sync_copy(hb_hbm.at[pl.ds(sig, 1)], hb_v)

        def init_body(i, c):
            off = i * 16
            mv_v[pl.ds(off, 16)] = jnp.ones((16,), jnp.float32)
            xrow_v[0, pl.ds(off, 16)] = jnp.zeros((16,), jnp.float32)
            return c
        lax.fori_loop(0, NA // 16, init_body, 0, unroll=False)

        l_rows = []
        lt_rows = []
        diag_l = []
        ys = []
        yvec = jnp.zeros((16,), jnp.float32)
        xvec = jnp.zeros((16,), jnp.float32)
        xs = [jnp.zeros((16,), jnp.float32)] * KMAX
        ivec = jnp.zeros((16,), jnp.int32)

        for k in range(KMAX):
            xs_prev = xs

            xb_prev = [_bf16_round(x) for x in xs_prev[:k]]

            def sweep_body(i, c, k=k, xb_prev=xb_prev):
                bmax, bidx = c
                off = i * 16
                hv = hb_v[0, pl.ds(off, 16)]
                if k > 0:
                    bv = xb_prev[0] * rrows_v[0][0, pl.ds(off, 16)]
                    for j in range(1, k):
                        bv = bv + xb_prev[j] * rrows_v[j][0, pl.ds(off, 16)]
                    hv = hv - bv
                a = jnp.abs(hv) * mv_v[pl.ds(off, 16)]
                iv = lane + off
                take = a > bmax
                return (jnp.where(take, a, bmax), jnp.where(take, iv, bidx))

            bmax, bidx = lax.fori_loop(
                0, NA // 16, sweep_body,
                (jnp.full((16,), -1.0, jnp.float32),
                 jnp.zeros((16,), jnp.int32)), unroll=False)
            m = _red(bmax, jnp.maximum)
            cand = jnp.where(bmax == m, bidx, jnp.int32(NA))
            idx_splat = _red(cand, jnp.minimum)
            ivec = jnp.where(lane == k, idx_splat, ivec)
            lane0 = lane == 0
            plsc.store_scatter(mv_v, [idx_splat], jnp.zeros((16,), jnp.float32),
                               mask=lane0)

            plsc.store_scatter(idx1_v, [jnp.zeros((16,), jnp.int32)],
                               idx_splat, mask=lane0)
            cp = pltpu.async_copy(g_hbm.at[idx1_v], rows_v[k], gsem)

            if k == 0:
                e0 = jnp.where(lane == 0, jnp.float32(1.0), jnp.float32(0.0))
                l_rows = [e0]
                lt_rows = [e0]
                diag_l = [jnp.ones((16,), jnp.float32)]
            else:
                w = jnp.zeros((16,), jnp.float32)
                wjs = []
                for j in range(k):
                    gj = plsc.load_gather(
                        rows_v[j], [jnp.zeros((16,), jnp.int32), idx_splat])
                    dot = _red(l_rows[j] * w, jnp.add)
                    wj = (gj - dot) / diag_l[j]
                    wjs.append(wj)
                    w = jnp.where(lane == j, wj, w)
                diag_g = plsc.load_gather(diag_v, [idx_splat])
                corner = _newton_sqrt(
                    jnp.maximum(diag_g - _red(w * w, jnp.add),
                                jnp.float32(DIAG_EPS)))
                l_rows.append(w + jnp.where(lane == k, corner,
                                            jnp.float32(0.0)))
                for j in range(k):
                    lt_rows[j] = jnp.where(lane == k, wjs[j], lt_rows[j])
                lt_rows.append(jnp.where(lane == k, corner, jnp.float32(0.0)))
                diag_l.append(corner)

            h_ik = plsc.load_gather(
                hb_v, [jnp.zeros((16,), jnp.int32), idx_splat])
            yk = (h_ik - _red(l_rows[k] * yvec, jnp.add)) / diag_l[k]
            yvec = jnp.where(lane == k, yk, yvec)
            ys.append(yk)

            xvec = jnp.zeros((16,), jnp.float32)
            xs = [jnp.zeros((16,), jnp.float32)] * KMAX
            for j in range(k, -1, -1):
                dot = _red(lt_rows[j] * xvec, jnp.add)
                xj = (ys[j] - dot) / diag_l[j]
                xs[j] = xj
                xvec = jnp.where(lane == j, xj, xvec)

            cp.wait()

            def rnd_body(i, c, k=k):
                off = i * 16
                rrows_v[k][0, pl.ds(off, 16)] = _bf16_round(
                    rows_v[k][0, pl.ds(off, 16)])
                return c
            lax.fori_loop(0, NA // 16, rnd_body, 0, unroll=False)

        plsc.store_scatter(xrow_v, [jnp.zeros((16,), jnp.int32), ivec],
                           xvec, mask=lane < KMAX)
        pltpu.sync_copy(xrow_v, x_hbm.at[pl.ds(sig, 1)])

        idx16_v[...] = ivec
        pltpu.async_copy(d_hbm.at[idx16_v], drows_v, gsem).wait()

        xb = [_bf16_round(x) for x in xs]

        def yp_body(i, c, xb=xb):
            off = i * 16
            acc = xb[0] * _bf16_round(drows_v[0, pl.ds(off, 16)])
            for j in range(1, KMAX):
                acc = acc + xb[j] * _bf16_round(drows_v[j, pl.ds(off, 16)])
            yp_v[0, pl.ds(off, 16)] = acc
            return c
        lax.fori_loop(0, NF // 16, yp_body, 0, unroll=False)
        pltpu.sync_copy(yp_v, yp_hbm.at[pl.ds(sig, 1)])
        return carry

    lax.fori_loop(0, PER_W, per_signal, 0, unroll=False)


def kernel(Y, D):
    A = jnp.concatenate([D, Y], axis=0)
    Z = _mm(A, D.T)
    G = Z[:NA]
    hb = Z[NA:]
    diag = jnp.diagonal(G)
    X, Yp = _omp_sc(hb, G, diag, D)
    return (Yp, X)
